# Initial kernel scaffold; baseline (speedup 1.0000x reference)
import jax
import jax.numpy as jnp
from jax.experimental import pallas as pl


def kernel(obj_feature, pos, query_points, params):
    raise NotImplementedError("write your pallas kernel here")


# speedup vs baseline: 3.1514x; 3.1514x over previous
import functools

import jax
import jax.numpy as jnp
from jax import lax
from jax.experimental import pallas as pl
from jax.experimental.pallas import tpu as pltpu
from jax.experimental.pallas import tpu_sc as plsc

N = 4096
DIM = 256
KNN = 16
POS_H = 64
DM = 1024
F32 = jnp.float32
HIGHEST = lax.Precision.HIGHEST


def _dot(a, b, dims, prec=HIGHEST):
    return lax.dot_general(a, b, dimension_numbers=(dims, ((), ())),
                           precision=prec, preferred_element_type=F32)


_BA = 256
_GA = N // _BA


def _proj_knn_body(objT, qpT, posT, posR,
                   Wso, bso, Wsq, bsq, Wk, bk, Wv, bv, Wq, bq,
                   q_out, v_out, key_out, idx_out):
    of = _dot(objT[...], Wso[...], ((1,), (1,))) + bso[...]
    key = _dot(of, Wk[...], ((1,), (1,))) + bk[...]
    val = _dot(of, Wv[...], ((1,), (1,))) + bv[...]
    qf = _dot(qpT[...], Wsq[...], ((1,), (1,))) + bsq[...]
    q = _dot(qf, Wq[...], ((1,), (1,))) + bq[...]
    q_out[...] = q
    v_out[...] = val
    key_out[...] = key

    pb = posT[...]
    prow = posR[...]
    sqc = jnp.sum(pb * pb, axis=1, keepdims=True)
    sqr = jnp.sum(prow * prow, axis=0, keepdims=True)
    pbb = pb.astype(jnp.bfloat16).astype(F32)
    prb = prow.astype(jnp.bfloat16).astype(F32)
    acc = pbb[:, 0:1] * prb[0:1, :]
    acc = acc + pbb[:, 1:2] * prb[1:2, :]
    acc = acc + pbb[:, 2:3] * prb[2:3, :]
    d = (sqc + sqr) - 2.0 * acc

    iota = lax.broadcasted_iota(jnp.int32, (_BA, N), 1)
    big = jnp.int32(2 ** 30)
    cols = []
    for _ in range(KNN):
        m = jnp.min(d, axis=1, keepdims=True)
        cand = jnp.where(d <= m, iota, big)
        sel = jnp.min(cand, axis=1, keepdims=True)
        cols.append(sel)
        d = jnp.where(iota == sel, jnp.float32(jnp.inf), d)
    idx_out[...] = jnp.concatenate(cols, axis=1)


def _proj_knn(objT, qpT, posT16, posR, p):
    blk = lambda r, c: pl.BlockSpec((r, c), lambda i: (i, 0))
    whole = lambda r, c: pl.BlockSpec((r, c), lambda i: (0, 0))
    return pl.pallas_call(
        _proj_knn_body,
        grid=(_GA,),
        in_specs=[
            blk(_BA, DIM), blk(_BA, DIM), blk(_BA, 16), whole(8, N),
            whole(DIM, DIM), whole(1, DIM), whole(DIM, DIM), whole(1, DIM),
            whole(DIM, DIM), whole(1, DIM), whole(DIM, DIM), whole(1, DIM),
            whole(DIM, DIM), whole(1, DIM),
        ],
        out_specs=[blk(_BA, DIM), blk(_BA, DIM), blk(_BA, DIM),
                   blk(_BA, KNN)],
        out_shape=[
            jax.ShapeDtypeStruct((N, DIM), F32),
            jax.ShapeDtypeStruct((N, DIM), F32),
            jax.ShapeDtypeStruct((N, DIM), F32),
            jax.ShapeDtypeStruct((N, KNN), jnp.int32),
        ],
    )(objT, qpT, posT16, posR,
      p['Wso'], p['bso2'], p['Wsq'], p['bsq2'], p['Wk'], p['bk2'],
      p['Wv'], p['bv2'], p['Wq'], p['bq2'])


_NW = 32
_CHUNK = 128
_ROWS = N * KNN
_PER_W = _ROWS // _NW
_NCH = _PER_W // _CHUNK


def _sc_gather(key, ptab, idx_flat):
    mesh = plsc.VectorSubcoreMesh(core_axis_name="c", subcore_axis_name="s")

    @functools.partial(
        pl.kernel,
        out_type=[jax.ShapeDtypeStruct((_ROWS, DIM), F32),
                  jax.ShapeDtypeStruct((_ROWS, 128), F32)],
        mesh=mesh,
        scratch_types=[
            pltpu.VMEM((_CHUNK,), jnp.int32),
            pltpu.VMEM((_CHUNK, DIM), F32),
            pltpu.VMEM((_CHUNK, 128), F32),
            pltpu.SemaphoreType.DMA,
            pltpu.SemaphoreType.DMA,
        ],
    )
    def k(key_hbm, ptab_hbm, idx_hbm, kg_hbm, pg_hbm,
          idx_v, rows_v, prow_v, sem1, sem2):
        wid = lax.axis_index("s") * 2 + lax.axis_index("c")

        def body(c, carry):
            base = wid * _PER_W + c * _CHUNK
            pltpu.sync_copy(idx_hbm.at[pl.ds(base, _CHUNK)], idx_v)
            cp1 = pltpu.async_copy(key_hbm.at[idx_v], rows_v, sem1)
            cp2 = pltpu.async_copy(ptab_hbm.at[idx_v], prow_v, sem2)
            cp1.wait()
            cp2.wait()
            pltpu.sync_copy(rows_v, kg_hbm.at[pl.ds(base, _CHUNK)])
            pltpu.sync_copy(prow_v, pg_hbm.at[pl.ds(base, _CHUNK)])
            return carry

        lax.fori_loop(0, _NCH, body, 0)

    return k(key, ptab, idx_flat)


_BC = 256
_GC = N // _BC


def _pe_stats_body(pg, posT, Wp1p, bp1, sh_out, shh_out):
    i = pl.program_id(0)
    pg3 = pg[...][:, :16].reshape(_BC, KNN, 16)
    pr = (posT[...][:, None, :] - pg3).reshape(_BC * KNN, 16)
    h1 = _dot(pr, Wp1p[...], ((1,), (1,))) + bp1[...]

    @pl.when(i == 0)
    def _():
        sh_out[...] = jnp.zeros_like(sh_out)
        shh_out[...] = jnp.zeros_like(shh_out)

    sh_out[...] += jnp.sum(h1, axis=0, keepdims=True)
    shh_out[...] += jnp.sum(h1 * h1, axis=0, keepdims=True)


def _pe_stats(pg, posT16, p):
    return pl.pallas_call(
        _pe_stats_body,
        grid=(_GC,),
        in_specs=[
            pl.BlockSpec((_BC * KNN, 128), lambda i: (i, 0)),
            pl.BlockSpec((_BC, 16), lambda i: (i, 0)),
            pl.BlockSpec((POS_H, 16), lambda i: (0, 0)),
            pl.BlockSpec((1, POS_H), lambda i: (0, 0)),
        ],
        out_specs=[pl.BlockSpec((1, POS_H), lambda i: (0, 0)),
                   pl.BlockSpec((1, POS_H), lambda i: (0, 0))],
        out_shape=[jax.ShapeDtypeStruct((1, POS_H), F32),
                   jax.ShapeDtypeStruct((1, POS_H), F32)],
    )(pg, posT16, p['Wp1p'], p['bp12'])


def _x_stats_body(pg, posT, q, kg, Wp1f, bp1f, Wp2, bp2, sx_out, sxx_out):
    i = pl.program_id(0)
    pg3 = pg[...][:, :16].reshape(_BC, KNN, 16)
    pr = (posT[...][:, None, :] - pg3).reshape(_BC * KNN, 16)
    h = jnp.maximum(_dot(pr, Wp1f[...], ((1,), (1,))) + bp1f[...], 0.0)
    pe = _dot(h, Wp2[...], ((1,), (1,))) + bp2[...]
    x3 = q[...][:, None, :] - kg[...].reshape(_BC, KNN, DIM)
    x = x3.reshape(_BC * KNN, DIM) + pe

    @pl.when(i == 0)
    def _():
        sx_out[...] = jnp.zeros_like(sx_out)
        sxx_out[...] = jnp.zeros_like(sxx_out)

    sx_out[...] += jnp.sum(x, axis=0, keepdims=True)
    sxx_out[...] += _dot(x, x, ((0,), (0,)))


def _x_stats(pg, posT16, q, kg, wp1f, bp1f, p):
    return pl.pallas_call(
        _x_stats_body,
        grid=(_GC,),
        in_specs=[
            pl.BlockSpec((_BC * KNN, 128), lambda i: (i, 0)),
            pl.BlockSpec((_BC, 16), lambda i: (i, 0)),
            pl.BlockSpec((_BC, DIM), lambda i: (i, 0)),
            pl.BlockSpec((_BC * KNN, DIM), lambda i: (i, 0)),
            pl.BlockSpec((POS_H, 16), lambda i: (0, 0)),
            pl.BlockSpec((1, POS_H), lambda i: (0, 0)),
            pl.BlockSpec((DIM, POS_H), lambda i: (0, 0)),
            pl.BlockSpec((1, DIM), lambda i: (0, 0)),
        ],
        out_specs=[pl.BlockSpec((1, DIM), lambda i: (0, 0)),
                   pl.BlockSpec((DIM, DIM), lambda i: (0, 0))],
        out_shape=[jax.ShapeDtypeStruct((1, DIM), F32),
                   jax.ShapeDtypeStruct((DIM, DIM), F32)],
    )(pg, posT16, q, kg, wp1f, bp1f, p['Wp2'], p['bp22'])


def _fold_a_body(sx, sxx, Wa1T, ba1r, gar, btar, w_out, b_out):
    m = jnp.float32(N * KNN)
    mx = sx[...] / m
    exx = sxx[...] / m
    cov = exx - _dot(mx, mx, ((0,), (0,)))
    w1cT = _dot(cov, Wa1T[...], ((1,), (0,)))
    var = jnp.sum(w1cT * Wa1T[...], axis=0, keepdims=True)
    mean_a = _dot(mx, Wa1T[...], ((1,), (0,))) + ba1r[...]
    scale = gar[...] * lax.rsqrt(var + 1e-5)
    w_out[...] = Wa1T[...] * scale
    b_out[...] = (ba1r[...] - mean_a) * scale + btar[...]


def _fold_a(sx, sxx, p):
    whole = lambda r, c: pl.BlockSpec((r, c), lambda: (0, 0))
    return pl.pallas_call(
        _fold_a_body,
        in_specs=[whole(1, DIM), whole(DIM, DIM), whole(DIM, DM),
                  whole(1, DM), whole(1, DM), whole(1, DM)],
        out_specs=[whole(DIM, DM), whole(1, DM)],
        out_shape=[jax.ShapeDtypeStruct((DIM, DM), F32),
                   jax.ShapeDtypeStruct((1, DM), F32)],
    )(sx, sxx, p['Wa1T'], p['ba1r'], p['gar'], p['btar'])


_BD = 128
_GD = N // _BD


def _attn_body(pg, posT, q, v, kg, Wp1f, bp1f, Wp2, bp2,
               Wa1f, ba1f, Wa2, ba2, We, be, y_out):
    pg3 = pg[...][:, :16].reshape(_BD, KNN, 16)
    pr = (posT[...][:, None, :] - pg3).reshape(_BD * KNN, 16)
    h = jnp.maximum(_dot(pr, Wp1f[...], ((1,), (1,))) + bp1f[...], 0.0)
    pe = _dot(h, Wp2[...], ((1,), (1,))) + bp2[...]
    x3 = q[...][:, None, :] - kg[...].reshape(_BD, KNN, DIM)
    x = x3.reshape(_BD * KNN, DIM) + pe
    a = jnp.maximum(_dot(x, Wa1f[...], ((1,), (0,))) + ba1f[...], 0.0)
    logits = _dot(a, Wa2[...], ((1,), (1,))) + ba2[...]
    l3 = logits.reshape(_BD, KNN, DIM)
    mx = jnp.max(l3, axis=1, keepdims=True)
    e = jnp.exp(l3 - mx)
    att = e / jnp.sum(e, axis=1, keepdims=True)
    val3 = v[...][:, None, :] + pe.reshape(_BD, KNN, DIM)
    agg = jnp.sum(att * val3, axis=1)
    y_out[...] = _dot(agg, We[...], ((1,), (1,))) + be[...]


def _attn(pg, posT16, q, v, kg, wp1f, bp1f, wa1f, ba1f_row, p):
    blk = lambda r, c: pl.BlockSpec((r, c), lambda i: (i, 0))
    whole = lambda r, c: pl.BlockSpec((r, c), lambda i: (0, 0))
    return pl.pallas_call(
        _attn_body,
        grid=(_GD,),
        in_specs=[
            blk(_BD * KNN, 128), blk(_BD, 16), blk(_BD, DIM), blk(_BD, DIM),
            blk(_BD * KNN, DIM),
            whole(POS_H, 16), whole(1, POS_H), whole(DIM, POS_H),
            whole(1, DIM),
            whole(DIM, DM), whole(1, DM), whole(DIM, DM), whole(1, DIM),
            whole(DIM, DIM), whole(1, DIM),
        ],
        out_specs=[blk(_BD, DIM)],
        out_shape=[jax.ShapeDtypeStruct((N, DIM), F32)],
    )(pg, posT16, q, v, kg, wp1f, bp1f, p['Wp2'], p['bp22'],
      wa1f, ba1f_row, p['Wa2'], p['ba22'], p['We'], p['be2'])[0]


def kernel(obj_feature, pos, query_points, params):
    p = dict(params)
    p['bso2'] = p['bso'][None, :]
    p['bsq2'] = p['bsq'][None, :]
    p['bk2'] = p['bk'][None, :]
    p['bv2'] = p['bv'][None, :]
    p['bq2'] = p['bq'][None, :]
    p['bp12'] = p['bp1'][None, :]
    p['bp22'] = p['bp2'][None, :]
    p['ba22'] = p['ba2'][None, :]
    p['be2'] = p['be'][None, :]
    p['ba1r'] = p['ba1'][None, :]
    p['gar'] = p['ga'][None, :]
    p['btar'] = p['bta'][None, :]
    p['Wa1T'] = jnp.transpose(p['Wa1'])
    p['Wp1p'] = jnp.zeros((POS_H, 16), F32).at[:, :3].set(p['Wp1'])

    objT = jnp.transpose(obj_feature[0])
    qpT = jnp.transpose(query_points[0])
    posT = jnp.transpose(pos[0])
    posT16 = jnp.zeros((N, 16), F32).at[:, :3].set(posT)
    posR = jnp.zeros((8, N), F32).at[:3, :].set(pos[0])

    ptab = jnp.zeros((N, 128), F32).at[:, :3].set(posT)

    q, v, key, idx = _proj_knn(objT, qpT, posT16, posR, p)
    kg, pg = _sc_gather(key, ptab, idx.reshape(-1))

    sh, shh = _pe_stats(pg, posT16, p)
    m = float(N * KNN)
    mh = sh / m
    vh = shh / m - mh * mh
    s_pe = p['gp'][None, :] * jax.lax.rsqrt(vh + 1e-5)
    wp1f = p['Wp1p'] * s_pe[0][:, None]
    bp1f = (p['bp1'][None, :] - mh) * s_pe + p['btp'][None, :]

    sx, sxx = _x_stats(pg, posT16, q, kg, wp1f, bp1f, p)
    wa1fT, ba1f = _fold_a(sx, sxx, p)

    y = _attn(pg, posT16, q, v, kg, wp1f, bp1f, wa1fT, ba1f, p)
    return jnp.transpose(y)[None]
